# Initial kernel scaffold; baseline (speedup 1.0000x reference)
import jax
import jax.numpy as jnp
from jax.experimental import pallas as pl


def kernel(user_table, item_table, eigs, lambda0_0, path_w0, lambda0_1, path_w1, indices0, path_type0, indices1, path_type1):
    raise NotImplementedError("write your pallas kernel here")


# speedup vs baseline: 8.3445x; 8.3445x over previous
import functools

import jax
import jax.numpy as jnp
from jax import lax
from jax.experimental import pallas as pl
from jax.experimental.pallas import tpu as pltpu
from jax.experimental.pallas import tpu_sc as plsc

N_USERS = 4000
N_ITEMS = 6000
N = N_USERS + N_ITEMS
H = 128
EIG = 16
E = 320000
NP = 6

NC = 2
NS = 16
NW = NC * NS
EPW = E // NW
B = 80
NBLK = EPW // B
ND = 10240
NSL = ND // NS
INV_SQRT_H = 1.0 / (128.0 ** 0.5)

_MESH = plsc.VectorSubcoreMesh(
    core_axis_name="c", subcore_axis_name="s", num_cores=NC, num_subcores=NS
)

_IOTA16 = None


def _iota16():
    return lax.iota(jnp.int32, 16)


_PITCH = 17


def _dot_group(b0, b1, g, d, q1d):
    nsub = d // 16
    gbase = 16 * _PITCH * g
    for e in range(16):
        row = 16 * g + e
        acc = b0[row, pl.ds(0, 16)] * b1[row, pl.ds(0, 16)]
        for k in range(1, nsub):
            acc = acc + b0[row, pl.ds(16 * k, 16)] * b1[row, pl.ds(16 * k, 16)]
        q1d[pl.ds(gbase + _PITCH * e, 16)] = acc
    ridx = _iota16() * _PITCH + gbase
    x = plsc.load_gather(q1d, [ridx])
    for j in range(1, 16):
        x = x + plsc.load_gather(q1d, [ridx + j])
    return x


def _sc_logits_kernel(
    y_hbm, g_hbm, i0_hbm, i1_hbm, pt_hbm, epw_hbm, lam_hbm,
    e0_out, ez_out, dpart_out,
    i0_v, i1_v, pt_v, y0_b, y1_b, g0_b, g1_b, q_b, q2_b, e0_b, ez_b,
    epw_v, lam_v, zden_v, d0_sp, d1_sp,
    sem0, sem1, sem2, sem3,
):
    cid = lax.axis_index("c")
    sid = lax.axis_index("s")
    wid = cid * NS + sid
    ebase = wid * EPW

    pltpu.sync_copy(epw_hbm, epw_v)
    pltpu.sync_copy(lam_hbm, lam_v)

    z16 = jnp.zeros((16,), jnp.float32)
    for k in range(NSL // 16):
        zden_v[pl.ds(16 * k, 16)] = z16
    pltpu.sync_copy(zden_v, d0_sp.at[pl.ds(sid * NSL, NSL)])
    pltpu.sync_copy(zden_v, d1_sp.at[pl.ds(sid * NSL, NSL)])
    plsc.subcore_barrier()

    lam = lam_v[...]

    @pl.loop(0, NBLK)
    def _block(b):
        base = ebase + b * B
        pltpu.sync_copy(i0_hbm.at[pl.ds(base, B)], i0_v)
        pltpu.sync_copy(i1_hbm.at[pl.ds(base, B)], i1_v)
        pltpu.sync_copy(pt_hbm.at[pl.ds(base, B)], pt_v)
        c0 = pltpu.async_copy(y_hbm.at[i0_v], y0_b, sem0)
        c1 = pltpu.async_copy(y_hbm.at[i1_v], y1_b, sem1)
        c2 = pltpu.async_copy(g_hbm.at[i0_v], g0_b, sem2)
        c3 = pltpu.async_copy(g_hbm.at[i1_v], g1_b, sem3)
        c0.wait()
        c1.wait()
        c2.wait()
        c3.wait()

        for g in range(B // 16):
            x = _dot_group(y0_b, y1_b, g, H, q_b)
            yv = _dot_group(g0_b, g1_b, g, EIG, q2_b)
            s0 = x * INV_SQRT_H + lam * yv
            e0_b[pl.ds(16 * g, 16)] = jnp.exp(s0)
            ptg = pt_v[pl.ds(16 * g, 16)]
            ez_b[pl.ds(16 * g, 16)] = plsc.load_gather(epw_v, [ptg])

        pltpu.sync_copy(e0_b, e0_out.at[pl.ds(base, B)])
        pltpu.sync_copy(ez_b, ez_out.at[pl.ds(base, B)])
        pltpu.sync_copy(e0_b, d0_sp.at[i0_v], add=True)
        pltpu.sync_copy(ez_b, d1_sp.at[i0_v], add=True)

    plsc.subcore_barrier()
    pltpu.sync_copy(
        d0_sp.at[pl.ds(sid * NSL, NSL)], dpart_out.at[cid, 0, pl.ds(sid * NSL, NSL)]
    )
    pltpu.sync_copy(
        d1_sp.at[pl.ds(sid * NSL, NSL)], dpart_out.at[cid, 1, pl.ds(sid * NSL, NSL)]
    )


def _sc_aggregate_kernel(
    y_hbm, i0_hbm, i1_hbm, e0_hbm, ez_hbm, dpart_hbm,
    opart_out,
    i0_v, i1_v, e0_v, ez_v, vrows, d0_t, d1_t, dtmp, zb, out_sp,
    sem0, sem1,
):
    cid = lax.axis_index("c")
    sid = lax.axis_index("s")
    wid = cid * NS + sid
    ebase = wid * EPW

    pltpu.sync_copy(dpart_hbm.at[0, 0], d0_t)
    pltpu.sync_copy(dpart_hbm.at[1, 0], dtmp)

    @pl.loop(0, ND // 16)
    def _add0(k):
        d0_t[pl.ds(k * 16, 16)] = d0_t[pl.ds(k * 16, 16)] + dtmp[pl.ds(k * 16, 16)]

    pltpu.sync_copy(dpart_hbm.at[0, 1], d1_t)
    pltpu.sync_copy(dpart_hbm.at[1, 1], dtmp)

    @pl.loop(0, ND // 16)
    def _add1(k):
        d1_t[pl.ds(k * 16, 16)] = d1_t[pl.ds(k * 16, 16)] + dtmp[pl.ds(k * 16, 16)]

    z16 = jnp.zeros((16,), jnp.float32)
    for r in range(16):
        for k in range(H // 16):
            zb[r, pl.ds(16 * k, 16)] = z16

    @pl.loop(0, NSL // 16)
    def _zero(t):
        pltpu.sync_copy(zb, out_sp.at[pl.ds(sid * NSL + t * 16, 16), :])

    plsc.subcore_barrier()

    @pl.loop(0, NBLK)
    def _block(b):
        base = ebase + b * B
        pltpu.sync_copy(i0_hbm.at[pl.ds(base, B)], i0_v)
        pltpu.sync_copy(i1_hbm.at[pl.ds(base, B)], i1_v)
        pltpu.sync_copy(e0_hbm.at[pl.ds(base, B)], e0_v)
        pltpu.sync_copy(ez_hbm.at[pl.ds(base, B)], ez_v)
        cg = pltpu.async_copy(y_hbm.at[i1_v], vrows, sem0)
        cg.wait()

        for g in range(B // 16):
            i0g = i0_v[pl.ds(16 * g, 16)]
            d0g = plsc.load_gather(d0_t, [i0g])
            d1g = plsc.load_gather(d1_t, [i0g])
            e0g = e0_v[pl.ds(16 * g, 16)]
            ezg = ez_v[pl.ds(16 * g, 16)]
            sg = 0.5 * (e0g / d0g + ezg / d1g)
            for l in range(16):
                row = 16 * g + l
                bv = sg.at[jnp.full((16,), l, jnp.int32)].get(
                    mode="promise_in_bounds"
                )
                for k in range(H // 16):
                    vrows[row, pl.ds(16 * k, 16)] = (
                        vrows[row, pl.ds(16 * k, 16)] * bv
                    )

        cs = pltpu.async_copy(vrows, out_sp.at[i0_v], sem1, add=True)
        cs.wait()

    plsc.subcore_barrier()

    @pl.loop(0, NSL // 16)
    def _out(t):
        pltpu.sync_copy(
            out_sp.at[pl.ds(sid * NSL + t * 16, 16), :],
            opart_out.at[cid, pl.ds(sid * NSL + t * 16, 16), :],
        )


def _sc_logits(y, eigs, i0, i1, pt, epw, lam):
    return pl.kernel(
        _sc_logits_kernel,
        out_type=(
            jax.ShapeDtypeStruct((E,), jnp.float32),
            jax.ShapeDtypeStruct((E,), jnp.float32),
            jax.ShapeDtypeStruct((NC, 2, ND), jnp.float32),
        ),
        mesh=_MESH,
        compiler_params=pltpu.CompilerParams(needs_layout_passes=False),
        scratch_types=(
            pltpu.VMEM((B,), jnp.int32),
            pltpu.VMEM((B,), jnp.int32),
            pltpu.VMEM((B,), jnp.int32),
            pltpu.VMEM((B, H), jnp.float32),
            pltpu.VMEM((B, H), jnp.float32),
            pltpu.VMEM((B, H), jnp.float32),
            pltpu.VMEM((B, H), jnp.float32),
            pltpu.VMEM(((B // 16) * 16 * _PITCH,), jnp.float32),
            pltpu.VMEM(((B // 16) * 16 * _PITCH,), jnp.float32),
            pltpu.VMEM((B,), jnp.float32),
            pltpu.VMEM((B,), jnp.float32),
            pltpu.VMEM((16,), jnp.float32),
            pltpu.VMEM((16,), jnp.float32),
            pltpu.VMEM((NSL,), jnp.float32),
            pltpu.VMEM_SHARED((ND,), jnp.float32),
            pltpu.VMEM_SHARED((ND,), jnp.float32),
            pltpu.SemaphoreType.DMA,
            pltpu.SemaphoreType.DMA,
            pltpu.SemaphoreType.DMA,
            pltpu.SemaphoreType.DMA,
        ),
    )(y, eigs, i0, i1, pt, epw, lam)


def _sc_aggregate(y, i0, i1, e0a, eza, dpart):
    return pl.kernel(
        _sc_aggregate_kernel,
        out_type=jax.ShapeDtypeStruct((NC, ND, H), jnp.float32),
        mesh=_MESH,
        compiler_params=pltpu.CompilerParams(needs_layout_passes=False),
        scratch_types=(
            pltpu.VMEM((B,), jnp.int32),
            pltpu.VMEM((B,), jnp.int32),
            pltpu.VMEM((B,), jnp.float32),
            pltpu.VMEM((B,), jnp.float32),
            pltpu.VMEM((B, H), jnp.float32),
            pltpu.VMEM((ND,), jnp.float32),
            pltpu.VMEM((ND,), jnp.float32),
            pltpu.VMEM((ND,), jnp.float32),
            pltpu.VMEM((16, H), jnp.float32),
            pltpu.VMEM_SHARED((ND, H), jnp.float32),
            pltpu.SemaphoreType.DMA,
            pltpu.SemaphoreType.DMA,
        ),
    )(y, i0, i1, e0a, eza, dpart)


_TCB = 1000


def _ln_block(x):
    mu = jnp.mean(x, axis=-1, keepdims=True)
    var = jnp.mean((x - mu) ** 2, axis=-1, keepdims=True)
    return (x - mu) / jnp.sqrt(var + 1e-5)


def _tc_ln_kernel(x_ref, y_ref):
    y_ref[...] = _ln_block(x_ref[...])


def _tc_ln(x):
    grid = N // _TCB
    return pl.pallas_call(
        _tc_ln_kernel,
        grid=(grid,),
        in_specs=[pl.BlockSpec((_TCB, H), lambda i: (i, 0))],
        out_specs=pl.BlockSpec((_TCB, H), lambda i: (i, 0)),
        out_shape=jax.ShapeDtypeStruct((N, H), jnp.float32),
    )(x)


def _tc_comb_kernel(p0_ref, p1_ref, emb_ref, y_ref):
    e = p0_ref[...] + p1_ref[...]
    emb_ref[...] = e
    y_ref[...] = _ln_block(e)


def _tc_comb(p0, p1):
    grid = N // _TCB
    bs = pl.BlockSpec((_TCB, H), lambda i: (i, 0))
    return pl.pallas_call(
        _tc_comb_kernel,
        grid=(grid,),
        in_specs=[bs, bs],
        out_specs=(bs, bs),
        out_shape=(
            jax.ShapeDtypeStruct((N, H), jnp.float32),
            jax.ShapeDtypeStruct((N, H), jnp.float32),
        ),
    )(p0, p1)


def _tc_final_kernel(p0_ref, p1_ref, e0_ref, e1_ref, o_ref):
    o_ref[...] = (
        p0_ref[...] + p1_ref[...] + e0_ref[...] + e1_ref[...]
    ) * (1.0 / 3.0)


def _tc_final(p0, p1, emb0, emb1):
    grid = N // _TCB
    bs = pl.BlockSpec((_TCB, H), lambda i: (i, 0))
    return pl.pallas_call(
        _tc_final_kernel,
        grid=(grid,),
        in_specs=[bs, bs, bs, bs],
        out_specs=bs,
        out_shape=jax.ShapeDtypeStruct((N, H), jnp.float32),
    )(p0, p1, emb0, emb1)


def kernel(user_table, item_table, eigs, lambda0_0, path_w0, lambda0_1,
           path_w1, indices0, path_type0, indices1, path_type1):
    emb0 = jnp.concatenate([user_table, item_table], axis=0)
    eigs_p = jnp.zeros((N, H), jnp.float32).at[:, :EIG].set(
        eigs.astype(jnp.float32)
    )

    def prep(indices, path_type, lam0, path_w):
        i0 = indices[0].astype(jnp.int32)
        i1 = indices[1].astype(jnp.int32)
        pt = path_type.astype(jnp.int32)
        epw = jnp.zeros((16,), jnp.float32).at[:NP].set(
            jnp.exp(path_w.reshape(-1))
        )
        lam = jnp.full((16,), jnp.exp(lam0[0]), jnp.float32)
        return i0, i1, pt, epw, lam

    layers = (
        prep(indices0, path_type0, lambda0_0, path_w0),
        prep(indices1, path_type1, lambda0_1, path_w1),
    )

    y = _tc_ln(emb0)
    i0, i1, pt, epw, lam = layers[0]
    e0a, eza, dpart = _sc_logits(y, eigs_p, i0, i1, pt, epw, lam)
    opart = _sc_aggregate(y, i0, i1, e0a, eza, dpart)
    emb1, y = _tc_comb(opart[0, :N], opart[1, :N])
    i0, i1, pt, epw, lam = layers[1]
    e0a, eza, dpart = _sc_logits(y, eigs_p, i0, i1, pt, epw, lam)
    opart = _sc_aggregate(y, i0, i1, e0a, eza, dpart)
    return _tc_final(opart[0, :N], opart[1, :N], emb0, emb1)
